# baseline (speedup 1.0000x reference)
import functools

import jax
import jax.numpy as jnp
from jax import lax
from jax.experimental import pallas as pl
from jax.experimental.pallas import tpu as pltpu
from jax.experimental.pallas import tpu_sc as plsc

EMBED_DIM = 128


@functools.cache
def _build_sc_lookup():
    mesh = plsc.VectorSubcoreMesh(core_axis_name="c", subcore_axis_name="s")

    @functools.partial(
        pl.kernel,
        mesh=mesh,
        out_type=jax.ShapeDtypeStruct((1, EMBED_DIM), jnp.float32),
        scratch_types=[
            pltpu.VMEM((1,), jnp.int32),
            pltpu.VMEM((1, EMBED_DIM), jnp.float32),
            pltpu.SemaphoreType.DMA,
        ],
    )
    def _sc_lookup(idx_hbm, table_hbm, out_hbm, idx_v, row_v, sem):
        cid = lax.axis_index("c")
        sid = lax.axis_index("s")

        @pl.when(jnp.logical_and(cid == 0, sid == 0))
        def _():
            pltpu.sync_copy(idx_hbm, idx_v)
            pltpu.async_copy(table_hbm.at[idx_v], row_v, sem).wait()
            pltpu.sync_copy(row_v, out_hbm)

    return _sc_lookup


def kernel(task_id, embedding_weight):
    idx = task_id.reshape(-1)[:1].astype(jnp.int32)
    out = _build_sc_lookup()(idx, embedding_weight)
    return out.reshape(EMBED_DIM)


# speedup vs baseline: 1.1589x; 1.1589x over previous
import functools

import jax
import jax.numpy as jnp
from jax import lax
from jax.experimental import pallas as pl
from jax.experimental.pallas import tpu as pltpu
from jax.experimental.pallas import tpu_sc as plsc

EMBED_DIM = 128


@functools.cache
def _build_sc_lookup():
    mesh = plsc.ScalarSubcoreMesh(axis_name="c", num_cores=1)

    @functools.partial(
        pl.kernel,
        mesh=mesh,
        out_type=jax.ShapeDtypeStruct((1, EMBED_DIM), jnp.float32),
        scratch_types=[
            pltpu.SMEM((1,), jnp.int32),
        ],
    )
    def _sc_lookup(idx_hbm, table_hbm, out_hbm, idx_s):
        pltpu.sync_copy(idx_hbm, idx_s)
        i = idx_s[0]
        pltpu.sync_copy(table_hbm.at[pl.ds(i, 1)], out_hbm)

    return _sc_lookup


def kernel(task_id, embedding_weight):
    idx = task_id.reshape(-1)[:1].astype(jnp.int32)
    out = _build_sc_lookup()(idx, embedding_weight)
    return out.reshape(EMBED_DIM)


# speedup vs baseline: 10.6509x; 9.1904x over previous
import jax
import jax.numpy as jnp
from jax.experimental import pallas as pl
from jax.experimental.pallas import tpu as pltpu

EMBED_DIM = 128


def _copy_row(idx_ref, row_ref, out_ref):
    del idx_ref
    out_ref[...] = row_ref[0]


def kernel(task_id, embedding_weight):
    idx = task_id.reshape(-1)[:1].astype(jnp.int32)
    table3 = embedding_weight.reshape(-1, 1, EMBED_DIM)
    grid_spec = pltpu.PrefetchScalarGridSpec(
        num_scalar_prefetch=1,
        grid=(1,),
        in_specs=[
            pl.BlockSpec((1, 1, EMBED_DIM), lambda i, idx_ref: (idx_ref[0], 0, 0)),
        ],
        out_specs=pl.BlockSpec((1, EMBED_DIM), lambda i, idx_ref: (0, 0)),
    )
    out = pl.pallas_call(
        _copy_row,
        grid_spec=grid_spec,
        out_shape=jax.ShapeDtypeStruct((1, EMBED_DIM), jnp.float32),
    )(idx, table3)
    return out.reshape(EMBED_DIM)


# speedup vs baseline: 12.2913x; 1.1540x over previous
import jax
import jax.numpy as jnp
from jax.experimental import pallas as pl
from jax.experimental.pallas import tpu as pltpu

EMBED_DIM = 128


def _lookup(idx_ref, table_ref, out_ref, sem):
    i = idx_ref[0]
    cp = pltpu.make_async_copy(table_ref.at[pl.ds(i, 1)], out_ref, sem)
    cp.start()
    cp.wait()


def kernel(task_id, embedding_weight):
    idx = task_id.reshape(-1)[:1].astype(jnp.int32)
    out = pl.pallas_call(
        _lookup,
        in_specs=[
            pl.BlockSpec(memory_space=pltpu.MemorySpace.SMEM),
            pl.BlockSpec(memory_space=pltpu.MemorySpace.HBM),
        ],
        out_specs=pl.BlockSpec(memory_space=pltpu.MemorySpace.HBM),
        out_shape=jax.ShapeDtypeStruct((1, EMBED_DIM), jnp.float32),
        scratch_shapes=[pltpu.SemaphoreType.DMA],
    )(idx, embedding_weight)
    return out.reshape(EMBED_DIM)


# speedup vs baseline: 12.5652x; 1.0223x over previous
import jax
import jax.numpy as jnp
from jax.experimental import pallas as pl
from jax.experimental.pallas import tpu as pltpu

EMBED_DIM = 128


def _lookup(idx_ref, table_ref, out_ref, sem):
    i = idx_ref[0]
    cp = pltpu.make_async_copy(table_ref.at[i], out_ref, sem)
    cp.start()
    cp.wait()


def kernel(task_id, embedding_weight):
    idx = task_id.reshape(-1)[:1].astype(jnp.int32)
    return pl.pallas_call(
        _lookup,
        in_specs=[
            pl.BlockSpec(memory_space=pltpu.MemorySpace.SMEM),
            pl.BlockSpec(memory_space=pltpu.MemorySpace.HBM),
        ],
        out_specs=pl.BlockSpec(memory_space=pltpu.MemorySpace.HBM),
        out_shape=jax.ShapeDtypeStruct((EMBED_DIM,), jnp.float32),
        scratch_shapes=[pltpu.SemaphoreType.DMA],
    )(idx, embedding_weight)
